# Initial kernel scaffold; baseline (speedup 1.0000x reference)
import jax
import jax.numpy as jnp
from jax.experimental import pallas as pl


def kernel(x, edge_index, batch, W0, b0, W1, b1, hop_att0, hop_atts, hop_biases, conv_atts, Wh, bh):
    raise NotImplementedError("write your pallas kernel here")


# speedup vs baseline: 27.5560x; 27.5560x over previous
import functools

import jax
import jax.numpy as jnp
from jax import lax
from jax.experimental import pallas as pl
from jax.experimental.pallas import tpu as pltpu
from jax.experimental.pallas import tpu_sc as plsc

N = 50000
E = 800000
NUM_GRAPHS = 64
IN_CH = 128
HID = 64
HEAD = 4
DH = 16
LAYER_NUM = 10
LAMBD = 1.0

NC = 2
NS = 16
NW = NC * NS
G = 128
EPW = 25600
NGW = EPW // G
EP = NW * EPW
NPAD = 51200
RPT = NPAD // NS
RB = 1000
NBLK = N // RB

import math

_LOGK = [math.log(LAMBD / k + 1.0 + 1e-6) for k in range(1, LAYER_NUM + 1)]


def _iota16():
    return jnp.arange(16, dtype=jnp.int32)


def _elu(v):
    return jnp.where(v > 0, v, jnp.exp(v) - 1.0)


def _sc_softplus(s):
    t = jnp.exp(-jnp.abs(s))
    u = 1.0 + t
    w = t / (u + 1.0)
    w2 = w * w
    poly = 1.0 / 3.0 + w2 * (1.0 / 5.0 + w2 * (1.0 / 7.0 + w2 * (1.0 / 9.0)))
    ln_u = 2.0 * w * (1.0 + w2 * poly)
    return jnp.maximum(s, 0.0) + ln_u


def _sc1_body(src_h, dst_h, zs_h, conv_h, zeros2_h,
              aij_h, adj_h,
              gsrc, gdst, dsc, rs, rd, convv, av, adj_s, sem):
    c = lax.axis_index("c")
    s = lax.axis_index("s")
    w = s * NC + c
    cn = c * N

    pltpu.sync_copy(zeros2_h.at[pl.ds(s * RPT, RPT)], adj_s.at[pl.ds(s * RPT, RPT)])
    pltpu.sync_copy(conv_h.at[c], convv)
    plsc.subcore_barrier()

    def group(g, carry):
        base = w * EPW + g * G
        pltpu.sync_copy(src_h.at[pl.ds(base, G)], gsrc)
        pltpu.sync_copy(dst_h.at[pl.ds(base, G)], dsc)
        for k in range(G // 16):
            sl = pl.ds(k * 16, 16)
            gsrc[sl] = gsrc[sl] + cn
            gdst[sl] = dsc[sl] + cn
        cp1 = pltpu.async_copy(zs_h.at[gsrc], rs, sem)
        cp2 = pltpu.async_copy(zs_h.at[gdst], rd, sem)
        cp1.wait()
        cp2.wait()
        for k in range(G // 16):
            rows = _iota16() + (k * 16)
            eid = _iota16() + (base + k * 16)
            mask = eid < E
            acc0 = jnp.zeros((16,), jnp.float32)
            acc1 = jnp.zeros((16,), jnp.float32)
            for ch in range(32):
                col = jnp.full((16,), ch, jnp.int32)
                v = plsc.load_gather(rs, [rows, col]) + plsc.load_gather(rd, [rows, col])
                ev = jnp.where(v > 0, v, jnp.exp(v) - 1.0)
                contrib = ev * convv[ch, :]
                if ch < 16:
                    acc0 = acc0 + contrib
                else:
                    acc1 = acc1 + contrib
            a0 = jnp.where(mask, _sc_softplus(acc0) + 1e-6, 0.0)
            a1 = jnp.where(mask, _sc_softplus(acc1) + 1e-6, 0.0)
            plsc.store_scatter(av, [rows, jnp.zeros((16,), jnp.int32)], a0)
            plsc.store_scatter(av, [rows, jnp.ones((16,), jnp.int32)], a1)
        pltpu.sync_copy(av, aij_h.at[pl.ds(c * EP + base, G)])
        pltpu.sync_copy(av, adj_s.at[dsc], add=True)
        return carry

    lax.fori_loop(0, NGW, group, 0)
    plsc.subcore_barrier()
    pltpu.sync_copy(adj_s.at[pl.ds(s * RPT, RPT)], adj_h.at[c, pl.ds(s * RPT, RPT)])


_SC_PARAMS = pltpu.CompilerParams(needs_layout_passes=False, use_tc_tiling_on_sc=False)


def _sc_pass1(srcp, dstp, zs, conv, zeros2):
    mesh = plsc.VectorSubcoreMesh(core_axis_name="c", subcore_axis_name="s")
    f = pl.kernel(
        _sc1_body,
        compiler_params=_SC_PARAMS,
        out_type=(
            jax.ShapeDtypeStruct((2 * EP, 2), jnp.float32),
            jax.ShapeDtypeStruct((NC, NPAD, 2), jnp.float32),
        ),
        mesh=mesh,
        scratch_types=(
            pltpu.VMEM((G,), jnp.int32),
            pltpu.VMEM((G,), jnp.int32),
            pltpu.VMEM((G,), jnp.int32),
            pltpu.VMEM((G, 32), jnp.float32),
            pltpu.VMEM((G, 32), jnp.float32),
            pltpu.VMEM((32, 16), jnp.float32),
            pltpu.VMEM((G, 2), jnp.float32),
            pltpu.VMEM_SHARED((NPAD, 2), jnp.float32),
            pltpu.SemaphoreType.DMA,
        ),
    )
    return f(srcp, dstp, zs, conv, zeros2)


def _sc2_body(src_h, dst_h, xs_h, aij_h, zeros32_h,
              out_h,
              gsrc, dsc, rx, ar, msg, out_s, sem):
    c = lax.axis_index("c")
    s = lax.axis_index("s")
    w = s * NC + c
    cn = c * N

    pltpu.sync_copy(zeros32_h.at[pl.ds(s * RPT, RPT)], out_s.at[pl.ds(s * RPT, RPT)])
    plsc.subcore_barrier()

    def group(g, carry):
        base = w * EPW + g * G
        pltpu.sync_copy(src_h.at[pl.ds(base, G)], gsrc)
        pltpu.sync_copy(dst_h.at[pl.ds(base, G)], dsc)
        for k in range(G // 16):
            sl = pl.ds(k * 16, 16)
            gsrc[sl] = gsrc[sl] + cn
        cp1 = pltpu.async_copy(xs_h.at[gsrc], rx, sem)
        pltpu.sync_copy(aij_h.at[pl.ds(c * EP + base, G)], ar)
        cp1.wait()
        for k in range(G // 16):
            rows = _iota16() + (k * 16)
            a0 = plsc.load_gather(ar, [rows, jnp.zeros((16,), jnp.int32)])
            a1 = plsc.load_gather(ar, [rows, jnp.ones((16,), jnp.int32)])
            for ch in range(32):
                col = jnp.full((16,), ch, jnp.int32)
                xv = plsc.load_gather(rx, [rows, col])
                m = xv * (a0 if ch < 16 else a1)
                plsc.store_scatter(msg, [rows, col], m)
        pltpu.sync_copy(msg, out_s.at[dsc], add=True)
        return carry

    lax.fori_loop(0, NGW, group, 0)
    plsc.subcore_barrier()
    pltpu.sync_copy(out_s.at[pl.ds(s * RPT, RPT)], out_h.at[c, pl.ds(s * RPT, RPT)])


def _sc_pass2(srcp, dstp, xs, aij, zeros32):
    mesh = plsc.VectorSubcoreMesh(core_axis_name="c", subcore_axis_name="s")
    f = pl.kernel(
        _sc2_body,
        compiler_params=_SC_PARAMS,
        out_type=jax.ShapeDtypeStruct((NC, NPAD, 32), jnp.float32),
        mesh=mesh,
        scratch_types=(
            pltpu.VMEM((G,), jnp.int32),
            pltpu.VMEM((G,), jnp.int32),
            pltpu.VMEM((G, 32), jnp.float32),
            pltpu.VMEM((G, 2), jnp.float32),
            pltpu.VMEM((G, 32), jnp.float32),
            pltpu.VMEM_SHARED((NPAD, 32), jnp.float32),
            pltpu.SemaphoreType.DMA,
        ),
    )
    return f(srcp, dstp, xs, aij, zeros32)


def _head_mat():
    ci = lax.broadcasted_iota(jnp.int32, (HID, HEAD), 0) // DH
    hi = lax.broadcasted_iota(jnp.int32, (HID, HEAD), 1)
    return (ci == hi).astype(jnp.float32)


def _t0_body(x_ref, w0_ref, b0_ref, w1_ref, b1_ref, ha0_ref, hb0_ref,
             xf_ref, z_ref, zs2_ref):
    S = _head_mat()
    h = jnp.dot(x_ref[...], w0_ref[...], preferred_element_type=jnp.float32) + b0_ref[...]
    h = _elu(h)
    h = jnp.dot(h, w1_ref[...], preferred_element_type=jnp.float32) + b1_ref[...]
    xf_ref[...] = h
    att = jnp.dot(_elu(h) * ha0_ref[...], S, preferred_element_type=jnp.float32) + hb0_ref[...]
    z = h * jnp.dot(att, S.T, preferred_element_type=jnp.float32)
    z_ref[...] = z
    zs = z * _LOGK[0]
    zs2_ref[0] = zs[:, :32]
    zs2_ref[1] = zs[:, 32:]


def _t0_call(x, W0, b0, W1, b1, ha0, hb0):
    return pl.pallas_call(
        _t0_body,
        grid=(NBLK,),
        in_specs=[
            pl.BlockSpec((RB, IN_CH), lambda i: (i, 0)),
            pl.BlockSpec((IN_CH, HID), lambda i: (0, 0)),
            pl.BlockSpec((1, HID), lambda i: (0, 0)),
            pl.BlockSpec((HID, HID), lambda i: (0, 0)),
            pl.BlockSpec((1, HID), lambda i: (0, 0)),
            pl.BlockSpec((1, HID), lambda i: (0, 0)),
            pl.BlockSpec((1, HEAD), lambda i: (0, 0)),
        ],
        out_specs=[
            pl.BlockSpec((RB, HID), lambda i: (i, 0)),
            pl.BlockSpec((RB, HID), lambda i: (i, 0)),
            pl.BlockSpec((2, RB, 32), lambda i: (0, i, 0)),
        ],
        out_shape=[
            jax.ShapeDtypeStruct((N, HID), jnp.float32),
            jax.ShapeDtypeStruct((N, HID), jnp.float32),
            jax.ShapeDtypeStruct((2, N, 32), jnp.float32),
        ],
    )(x, W0, b0, W1, b1, ha0, hb0)


def _t1_body(adj_ref, xf_ref, xs2_ref, isq_ref):
    S = _head_mat()
    adj = jnp.concatenate([adj_ref[0], adj_ref[1]], axis=1)
    isq = jax.lax.rsqrt(jnp.maximum(adj, 1e-32))
    isq_ref[...] = isq
    xs = xf_ref[...] * jnp.dot(isq, S.T, preferred_element_type=jnp.float32)
    xs2_ref[0] = xs[:, :32]
    xs2_ref[1] = xs[:, 32:]


def _t1_call(adj, xf):
    return pl.pallas_call(
        _t1_body,
        grid=(NBLK,),
        in_specs=[
            pl.BlockSpec((2, RB, 2), lambda i: (0, i, 0)),
            pl.BlockSpec((RB, HID), lambda i: (i, 0)),
        ],
        out_specs=[
            pl.BlockSpec((2, RB, 32), lambda i: (0, i, 0)),
            pl.BlockSpec((RB, HEAD), lambda i: (i, 0)),
        ],
        out_shape=[
            jax.ShapeDtypeStruct((2, N, 32), jnp.float32),
            jax.ShapeDtypeStruct((N, HEAD), jnp.float32),
        ],
    )(adj, xf)


def _t2_body(s2_ref, isq_ref, z_ref, hatA_ref, hatB_ref, hb_ref,
             xf_ref, zn_ref, zs2_ref, *, logk, logk_next):
    S = _head_mat()
    sfull = jnp.concatenate([s2_ref[0], s2_ref[1]], axis=1)
    xf = sfull * jnp.dot(isq_ref[...], S.T, preferred_element_type=jnp.float32)
    xf_ref[...] = xf
    z = z_ref[...]
    zs = z * logk
    hop = (jnp.dot(_elu(xf) * hatA_ref[...], S, preferred_element_type=jnp.float32)
           + jnp.dot(_elu(zs) * hatB_ref[...], S, preferred_element_type=jnp.float32)
           + hb_ref[...])
    zn = z + xf * jnp.dot(hop, S.T, preferred_element_type=jnp.float32)
    zn_ref[...] = zn
    zsn = zn * logk_next
    zs2_ref[0] = zsn[:, :32]
    zs2_ref[1] = zsn[:, 32:]


def _t2_call(s2, isq, z, hatA, hatB, hb, logk, logk_next):
    body = functools.partial(_t2_body, logk=logk, logk_next=logk_next)
    return pl.pallas_call(
        body,
        grid=(NBLK,),
        in_specs=[
            pl.BlockSpec((2, RB, 32), lambda i: (0, i, 0)),
            pl.BlockSpec((RB, HEAD), lambda i: (i, 0)),
            pl.BlockSpec((RB, HID), lambda i: (i, 0)),
            pl.BlockSpec((1, HID), lambda i: (0, 0)),
            pl.BlockSpec((1, HID), lambda i: (0, 0)),
            pl.BlockSpec((1, HEAD), lambda i: (0, 0)),
        ],
        out_specs=[
            pl.BlockSpec((RB, HID), lambda i: (i, 0)),
            pl.BlockSpec((RB, HID), lambda i: (i, 0)),
            pl.BlockSpec((2, RB, 32), lambda i: (0, i, 0)),
        ],
        out_shape=[
            jax.ShapeDtypeStruct((N, HID), jnp.float32),
            jax.ShapeDtypeStruct((N, HID), jnp.float32),
            jax.ShapeDtypeStruct((2, N, 32), jnp.float32),
        ],
    )(s2, isq, z, hatA, hatB, hb)


def _t3_body(z_ref, batch_ref, wh_ref, bh_ref, sums_ref, cnts_ref, y_ref):
    i = pl.program_id(0)
    zf = _elu(z_ref[...])
    b = batch_ref[0]
    oh = (lax.broadcasted_iota(jnp.int32, (NUM_GRAPHS, RB), 0) == b).astype(jnp.float32)
    psum = jnp.dot(oh, zf, preferred_element_type=jnp.float32)
    pcnt = jnp.dot(oh, jnp.ones((RB, HID), jnp.float32), preferred_element_type=jnp.float32)

    @pl.when(i == 0)
    def _init():
        sums_ref[...] = psum
        cnts_ref[...] = pcnt

    @pl.when(i > 0)
    def _acc():
        sums_ref[...] += psum
        cnts_ref[...] += pcnt

    @pl.when(i == NBLK - 1)
    def _fin():
        pooled = sums_ref[...] / jnp.maximum(cnts_ref[...], 1.0)
        y_ref[...] = jnp.dot(pooled, wh_ref[...], preferred_element_type=jnp.float32) + bh_ref[...]


def _t3_call(z, batch3, Wh, bh):
    outs = pl.pallas_call(
        _t3_body,
        grid=(NBLK,),
        in_specs=[
            pl.BlockSpec((RB, HID), lambda i: (i, 0)),
            pl.BlockSpec((1, 1, RB), lambda i: (i, 0, 0)),
            pl.BlockSpec((HID, 1), lambda i: (0, 0)),
            pl.BlockSpec((1, 1), lambda i: (0, 0)),
        ],
        out_specs=[
            pl.BlockSpec((NUM_GRAPHS, HID), lambda i: (0, 0)),
            pl.BlockSpec((NUM_GRAPHS, HID), lambda i: (0, 0)),
            pl.BlockSpec((NUM_GRAPHS, 1), lambda i: (0, 0)),
        ],
        out_shape=[
            jax.ShapeDtypeStruct((NUM_GRAPHS, HID), jnp.float32),
            jax.ShapeDtypeStruct((NUM_GRAPHS, HID), jnp.float32),
            jax.ShapeDtypeStruct((NUM_GRAPHS, 1), jnp.float32),
        ],
    )(z, batch3, Wh, bh)
    return outs[2]


def kernel(x, edge_index, batch, W0, b0, W1, b1, hop_att0, hop_atts,
           hop_biases, conv_atts, Wh, bh):
    pad = EP - E
    srcp = jnp.concatenate([edge_index[0], jnp.zeros((pad,), jnp.int32)])
    dstp = jnp.concatenate([edge_index[1], jnp.zeros((pad,), jnp.int32)])
    zeros2 = jnp.zeros((NPAD, 2), jnp.float32)
    zeros32 = jnp.zeros((NPAD, 32), jnp.float32)
    convs = jnp.broadcast_to(
        conv_atts.reshape(LAYER_NUM - 1, NC, 32, 1), (LAYER_NUM - 1, NC, 32, 16)
    ).astype(jnp.float32)

    b0r = b0.reshape(1, HID)
    b1r = b1.reshape(1, HID)
    ha0 = hop_att0.reshape(1, HID)
    hbs = hop_biases.reshape(LAYER_NUM, 1, HEAD)
    hatA = hop_atts[:, 0, :, :DH].reshape(LAYER_NUM - 1, 1, HID)
    hatB = hop_atts[:, 0, :, DH:].reshape(LAYER_NUM - 1, 1, HID)
    batch3 = batch.reshape(NBLK, 1, RB)

    xf, z, zs2 = _t0_call(x, W0, b0r, W1, b1r, ha0, hbs[0])
    for i in range(LAYER_NUM - 1):
        aij, adjp = _sc_pass1(srcp, dstp, zs2.reshape(2 * N, 32), convs[i], zeros2)
        adj = adjp[:, :N, :]
        xs2, isq = _t1_call(adj, xf)
        sp = _sc_pass2(srcp, dstp, xs2.reshape(2 * N, 32), aij, zeros32)
        s2 = sp[:, :N, :]
        logk_next = _LOGK[i + 1] if i + 1 < LAYER_NUM - 1 else _LOGK[-1]
        xf, z, zs2 = _t2_call(s2, isq, z, hatA[i], hatB[i], hbs[i + 1],
                              _LOGK[i], logk_next)
    return _t3_call(z, batch3, Wh, bh.reshape(1, 1))


# speedup vs baseline: 40.7976x; 1.4805x over previous
import functools

import jax
import jax.numpy as jnp
from jax import lax
from jax.experimental import pallas as pl
from jax.experimental.pallas import tpu as pltpu
from jax.experimental.pallas import tpu_sc as plsc

N = 50000
E = 800000
NUM_GRAPHS = 64
IN_CH = 128
HID = 64
HEAD = 4
DH = 16
LAYER_NUM = 10
LAMBD = 1.0

NC = 2
NS = 16
NW = NC * NS
G = 128
EPW = 25600
NGW = EPW // G
EP = NW * EPW
NPAD = 51200
RPT = NPAD // NS
RB = 1000
NBLK = N // RB

import math

_LOGK = [math.log(LAMBD / k + 1.0 + 1e-6) for k in range(1, LAYER_NUM + 1)]


def _iota16():
    return jnp.arange(16, dtype=jnp.int32)


def _elu(v):
    return jnp.where(v > 0, v, jnp.exp(v) - 1.0)


def _sc_softplus(s):
    t = jnp.exp(-jnp.abs(s))
    u = 1.0 + t
    w = t / (u + 1.0)
    w2 = w * w
    poly = 1.0 / 3.0 + w2 * (1.0 / 5.0 + w2 * (1.0 / 7.0 + w2 * (1.0 / 9.0)))
    ln_u = 2.0 * w * (1.0 + w2 * poly)
    return jnp.maximum(s, 0.0) + ln_u


IDXC = 20
NSG = NGW // IDXC


def _adjust_indices(raw, adj, cn, ngroups):
    def per_group(g, carry):
        for j in range(G // 16):
            sl = pl.ds(j * 16, 16)
            adj[g, sl] = raw[g, sl] + cn
        return carry
    lax.fori_loop(0, ngroups, per_group, 0)


def _sc1_body(src_h, dst_h, zs_h, conv_h, zeros2_h,
              aij_h, adj_h,
              idxs, idxd, idxr, rs2, rd2, convv, av, adj_s, sem):
    c = lax.axis_index("c")
    s = lax.axis_index("s")
    w = s * NC + c
    cn = c * N

    pltpu.sync_copy(zeros2_h.at[pl.ds(s * RPT, RPT)], adj_s.at[pl.ds(s * RPT, RPT)])
    pltpu.sync_copy(conv_h.at[c], convv)
    cvs = [convv[ch, :] for ch in range(32)]
    plsc.subcore_barrier()

    def supergroup(t, carry0):
        pltpu.sync_copy(src_h.at[w, pl.ds(t * IDXC, IDXC)], idxs)
        pltpu.sync_copy(dst_h.at[w, pl.ds(t * IDXC, IDXC)], idxr)
        _adjust_indices(idxs, idxs, cn, IDXC)
        _adjust_indices(idxr, idxd, cn, IDXC)
        pltpu.async_copy(zs_h.at[idxs.at[0]], rs2.at[0], sem.at[0])
        pltpu.async_copy(zs_h.at[idxd.at[0]], rd2.at[0], sem.at[0])

        def group(g, carry):
            p = lax.rem(g, 2)
            q = 1 - p

            @pl.when(g + 1 < IDXC)
            def _prefetch():
                pltpu.async_copy(zs_h.at[idxs.at[g + 1]], rs2.at[q], sem.at[q])
                pltpu.async_copy(zs_h.at[idxd.at[g + 1]], rd2.at[q], sem.at[q])

            pltpu.make_async_copy(zs_h.at[pl.ds(0, G)], rs2.at[p], sem.at[p]).wait()
            pltpu.make_async_copy(zs_h.at[pl.ds(0, G)], rd2.at[p], sem.at[p]).wait()
            pv = jnp.full((16,), p, jnp.int32)
            base = w * EPW + t * IDXC * G + g * G
            for k in range(G // 16):
                rows = _iota16() + (k * 16)
                eid = _iota16() + (base + k * 16)
                mask = eid < E
                acc0 = jnp.zeros((16,), jnp.float32)
                acc1 = jnp.zeros((16,), jnp.float32)
                for ch in range(32):
                    col = jnp.full((16,), ch, jnp.int32)
                    v = (plsc.load_gather(rs2, [pv, rows, col])
                         + plsc.load_gather(rd2, [pv, rows, col]))
                    ev = jnp.where(v > 0, v, jnp.exp(v) - 1.0)
                    contrib = ev * cvs[ch]
                    if ch < 16:
                        acc0 = acc0 + contrib
                    else:
                        acc1 = acc1 + contrib
                a0 = jnp.where(mask, _sc_softplus(acc0) + 1e-6, 0.0)
                a1 = jnp.where(mask, _sc_softplus(acc1) + 1e-6, 0.0)
                plsc.store_scatter(av, [rows, jnp.zeros((16,), jnp.int32)], a0)
                plsc.store_scatter(av, [rows, jnp.ones((16,), jnp.int32)], a1)
            pltpu.sync_copy(av, aij_h.at[pl.ds(c * EP + base, G)])
            pltpu.sync_copy(av, adj_s.at[idxr.at[g]], add=True)
            return carry

        lax.fori_loop(0, IDXC, group, 0)
        return carry0

    lax.fori_loop(0, NSG, supergroup, 0)
    plsc.subcore_barrier()
    pltpu.sync_copy(adj_s.at[pl.ds(s * RPT, RPT)], adj_h.at[c, pl.ds(s * RPT, RPT)])


_SC_PARAMS = pltpu.CompilerParams(needs_layout_passes=False, use_tc_tiling_on_sc=False)


def _sc_pass1(srcp, dstp, zs, conv, zeros2):
    mesh = plsc.VectorSubcoreMesh(core_axis_name="c", subcore_axis_name="s")
    f = pl.kernel(
        _sc1_body,
        compiler_params=_SC_PARAMS,
        out_type=(
            jax.ShapeDtypeStruct((2 * EP, 2), jnp.float32),
            jax.ShapeDtypeStruct((NC, NPAD, 2), jnp.float32),
        ),
        mesh=mesh,
        scratch_types=(
            pltpu.VMEM((IDXC, G), jnp.int32),
            pltpu.VMEM((IDXC, G), jnp.int32),
            pltpu.VMEM((IDXC, G), jnp.int32),
            pltpu.VMEM((2, G, 32), jnp.float32),
            pltpu.VMEM((2, G, 32), jnp.float32),
            pltpu.VMEM((32, 16), jnp.float32),
            pltpu.VMEM((G, 2), jnp.float32),
            pltpu.VMEM_SHARED((NPAD, 2), jnp.float32),
            pltpu.SemaphoreType.DMA((2,)),
        ),
    )
    return f(srcp, dstp, zs, conv, zeros2)


def _sc2_body(src_h, dst_h, xs_h, aij_h, zeros32_h,
              out_h,
              idxs, idxr, rx2, ar2, msg, out_s, sem):
    c = lax.axis_index("c")
    s = lax.axis_index("s")
    w = s * NC + c
    cn = c * N

    pltpu.sync_copy(zeros32_h.at[pl.ds(s * RPT, RPT)], out_s.at[pl.ds(s * RPT, RPT)])
    plsc.subcore_barrier()

    def supergroup(t, carry0):
        pltpu.sync_copy(src_h.at[w, pl.ds(t * IDXC, IDXC)], idxs)
        pltpu.sync_copy(dst_h.at[w, pl.ds(t * IDXC, IDXC)], idxr)
        _adjust_indices(idxs, idxs, cn, IDXC)
        sbase = w * EPW + t * IDXC * G
        pltpu.async_copy(xs_h.at[idxs.at[0]], rx2.at[0], sem.at[0])
        pltpu.async_copy(aij_h.at[pl.ds(c * EP + sbase, G)], ar2.at[0], sem.at[0])

        def group(g, carry):
            p = lax.rem(g, 2)
            q = 1 - p

            @pl.when(g + 1 < IDXC)
            def _prefetch():
                pltpu.async_copy(xs_h.at[idxs.at[g + 1]], rx2.at[q], sem.at[q])
                pltpu.async_copy(aij_h.at[pl.ds(c * EP + sbase + (g + 1) * G, G)],
                                 ar2.at[q], sem.at[q])

            pltpu.make_async_copy(xs_h.at[pl.ds(0, G)], rx2.at[p], sem.at[p]).wait()
            pltpu.make_async_copy(aij_h.at[pl.ds(0, G)], ar2.at[p], sem.at[p]).wait()
            pv = jnp.full((16,), p, jnp.int32)
            for k in range(G // 16):
                rows = _iota16() + (k * 16)
                a0 = plsc.load_gather(ar2, [pv, rows, jnp.zeros((16,), jnp.int32)])
                a1 = plsc.load_gather(ar2, [pv, rows, jnp.ones((16,), jnp.int32)])
                for ch in range(32):
                    col = jnp.full((16,), ch, jnp.int32)
                    xv = plsc.load_gather(rx2, [pv, rows, col])
                    m = xv * (a0 if ch < 16 else a1)
                    plsc.store_scatter(msg, [rows, col], m)
            pltpu.sync_copy(msg, out_s.at[idxr.at[g]], add=True)
            return carry

        lax.fori_loop(0, IDXC, group, 0)
        return carry0

    lax.fori_loop(0, NSG, supergroup, 0)
    plsc.subcore_barrier()
    pltpu.sync_copy(out_s.at[pl.ds(s * RPT, RPT)], out_h.at[c, pl.ds(s * RPT, RPT)])


def _sc_pass2(srcp, dstp, xs, aij, zeros32):
    mesh = plsc.VectorSubcoreMesh(core_axis_name="c", subcore_axis_name="s")
    f = pl.kernel(
        _sc2_body,
        compiler_params=_SC_PARAMS,
        out_type=jax.ShapeDtypeStruct((NC, NPAD, 32), jnp.float32),
        mesh=mesh,
        scratch_types=(
            pltpu.VMEM((IDXC, G), jnp.int32),
            pltpu.VMEM((IDXC, G), jnp.int32),
            pltpu.VMEM((2, G, 32), jnp.float32),
            pltpu.VMEM((2, G, 2), jnp.float32),
            pltpu.VMEM((G, 32), jnp.float32),
            pltpu.VMEM_SHARED((NPAD, 32), jnp.float32),
            pltpu.SemaphoreType.DMA((2,)),
        ),
    )
    return f(srcp, dstp, xs, aij, zeros32)


def _head_mat():
    ci = lax.broadcasted_iota(jnp.int32, (HID, HEAD), 0) // DH
    hi = lax.broadcasted_iota(jnp.int32, (HID, HEAD), 1)
    return (ci == hi).astype(jnp.float32)


def _t0_body(x_ref, w0_ref, b0_ref, w1_ref, b1_ref, ha0_ref, hb0_ref,
             xf_ref, z_ref, zs2_ref):
    S = _head_mat()
    h = jnp.dot(x_ref[...], w0_ref[...], preferred_element_type=jnp.float32) + b0_ref[...]
    h = _elu(h)
    h = jnp.dot(h, w1_ref[...], preferred_element_type=jnp.float32) + b1_ref[...]
    xf_ref[...] = h
    att = jnp.dot(_elu(h) * ha0_ref[...], S, preferred_element_type=jnp.float32) + hb0_ref[...]
    z = h * jnp.dot(att, S.T, preferred_element_type=jnp.float32)
    z_ref[...] = z
    zs = z * _LOGK[0]
    zs2_ref[0] = zs[:, :32]
    zs2_ref[1] = zs[:, 32:]


def _t0_call(x, W0, b0, W1, b1, ha0, hb0):
    return pl.pallas_call(
        _t0_body,
        grid=(NBLK,),
        in_specs=[
            pl.BlockSpec((RB, IN_CH), lambda i: (i, 0)),
            pl.BlockSpec((IN_CH, HID), lambda i: (0, 0)),
            pl.BlockSpec((1, HID), lambda i: (0, 0)),
            pl.BlockSpec((HID, HID), lambda i: (0, 0)),
            pl.BlockSpec((1, HID), lambda i: (0, 0)),
            pl.BlockSpec((1, HID), lambda i: (0, 0)),
            pl.BlockSpec((1, HEAD), lambda i: (0, 0)),
        ],
        out_specs=[
            pl.BlockSpec((RB, HID), lambda i: (i, 0)),
            pl.BlockSpec((RB, HID), lambda i: (i, 0)),
            pl.BlockSpec((2, RB, 32), lambda i: (0, i, 0)),
        ],
        out_shape=[
            jax.ShapeDtypeStruct((N, HID), jnp.float32),
            jax.ShapeDtypeStruct((N, HID), jnp.float32),
            jax.ShapeDtypeStruct((2, N, 32), jnp.float32),
        ],
    )(x, W0, b0, W1, b1, ha0, hb0)


def _t1_body(adj_ref, xf_ref, xs2_ref, isq_ref):
    S = _head_mat()
    adj = jnp.concatenate([adj_ref[0], adj_ref[1]], axis=1)
    isq = jax.lax.rsqrt(jnp.maximum(adj, 1e-32))
    isq_ref[...] = isq
    xs = xf_ref[...] * jnp.dot(isq, S.T, preferred_element_type=jnp.float32)
    xs2_ref[0] = xs[:, :32]
    xs2_ref[1] = xs[:, 32:]


def _t1_call(adj, xf):
    return pl.pallas_call(
        _t1_body,
        grid=(NBLK,),
        in_specs=[
            pl.BlockSpec((2, RB, 2), lambda i: (0, i, 0)),
            pl.BlockSpec((RB, HID), lambda i: (i, 0)),
        ],
        out_specs=[
            pl.BlockSpec((2, RB, 32), lambda i: (0, i, 0)),
            pl.BlockSpec((RB, HEAD), lambda i: (i, 0)),
        ],
        out_shape=[
            jax.ShapeDtypeStruct((2, N, 32), jnp.float32),
            jax.ShapeDtypeStruct((N, HEAD), jnp.float32),
        ],
    )(adj, xf)


def _t2_body(s2_ref, isq_ref, z_ref, hatA_ref, hatB_ref, hb_ref,
             xf_ref, zn_ref, zs2_ref, *, logk, logk_next):
    S = _head_mat()
    sfull = jnp.concatenate([s2_ref[0], s2_ref[1]], axis=1)
    xf = sfull * jnp.dot(isq_ref[...], S.T, preferred_element_type=jnp.float32)
    xf_ref[...] = xf
    z = z_ref[...]
    zs = z * logk
    hop = (jnp.dot(_elu(xf) * hatA_ref[...], S, preferred_element_type=jnp.float32)
           + jnp.dot(_elu(zs) * hatB_ref[...], S, preferred_element_type=jnp.float32)
           + hb_ref[...])
    zn = z + xf * jnp.dot(hop, S.T, preferred_element_type=jnp.float32)
    zn_ref[...] = zn
    zsn = zn * logk_next
    zs2_ref[0] = zsn[:, :32]
    zs2_ref[1] = zsn[:, 32:]


def _t2_call(s2, isq, z, hatA, hatB, hb, logk, logk_next):
    body = functools.partial(_t2_body, logk=logk, logk_next=logk_next)
    return pl.pallas_call(
        body,
        grid=(NBLK,),
        in_specs=[
            pl.BlockSpec((2, RB, 32), lambda i: (0, i, 0)),
            pl.BlockSpec((RB, HEAD), lambda i: (i, 0)),
            pl.BlockSpec((RB, HID), lambda i: (i, 0)),
            pl.BlockSpec((1, HID), lambda i: (0, 0)),
            pl.BlockSpec((1, HID), lambda i: (0, 0)),
            pl.BlockSpec((1, HEAD), lambda i: (0, 0)),
        ],
        out_specs=[
            pl.BlockSpec((RB, HID), lambda i: (i, 0)),
            pl.BlockSpec((RB, HID), lambda i: (i, 0)),
            pl.BlockSpec((2, RB, 32), lambda i: (0, i, 0)),
        ],
        out_shape=[
            jax.ShapeDtypeStruct((N, HID), jnp.float32),
            jax.ShapeDtypeStruct((N, HID), jnp.float32),
            jax.ShapeDtypeStruct((2, N, 32), jnp.float32),
        ],
    )(s2, isq, z, hatA, hatB, hb)


def _t3_body(z_ref, batch_ref, wh_ref, bh_ref, sums_ref, cnts_ref, y_ref):
    i = pl.program_id(0)
    zf = _elu(z_ref[...])
    b = batch_ref[0]
    oh = (lax.broadcasted_iota(jnp.int32, (NUM_GRAPHS, RB), 0) == b).astype(jnp.float32)
    psum = jnp.dot(oh, zf, preferred_element_type=jnp.float32)
    pcnt = jnp.dot(oh, jnp.ones((RB, HID), jnp.float32), preferred_element_type=jnp.float32)

    @pl.when(i == 0)
    def _init():
        sums_ref[...] = psum
        cnts_ref[...] = pcnt

    @pl.when(i > 0)
    def _acc():
        sums_ref[...] += psum
        cnts_ref[...] += pcnt

    @pl.when(i == NBLK - 1)
    def _fin():
        pooled = sums_ref[...] / jnp.maximum(cnts_ref[...], 1.0)
        y_ref[...] = jnp.dot(pooled, wh_ref[...], preferred_element_type=jnp.float32) + bh_ref[...]


def _t3_call(z, batch3, Wh, bh):
    outs = pl.pallas_call(
        _t3_body,
        grid=(NBLK,),
        in_specs=[
            pl.BlockSpec((RB, HID), lambda i: (i, 0)),
            pl.BlockSpec((1, 1, RB), lambda i: (i, 0, 0)),
            pl.BlockSpec((HID, 1), lambda i: (0, 0)),
            pl.BlockSpec((1, 1), lambda i: (0, 0)),
        ],
        out_specs=[
            pl.BlockSpec((NUM_GRAPHS, HID), lambda i: (0, 0)),
            pl.BlockSpec((NUM_GRAPHS, HID), lambda i: (0, 0)),
            pl.BlockSpec((NUM_GRAPHS, 1), lambda i: (0, 0)),
        ],
        out_shape=[
            jax.ShapeDtypeStruct((NUM_GRAPHS, HID), jnp.float32),
            jax.ShapeDtypeStruct((NUM_GRAPHS, HID), jnp.float32),
            jax.ShapeDtypeStruct((NUM_GRAPHS, 1), jnp.float32),
        ],
    )(z, batch3, Wh, bh)
    return outs[2]


def kernel(x, edge_index, batch, W0, b0, W1, b1, hop_att0, hop_atts,
           hop_biases, conv_atts, Wh, bh):
    pad = EP - E
    srcp = jnp.concatenate([edge_index[0], jnp.zeros((pad,), jnp.int32)]).reshape(NW, NGW, G)
    dstp = jnp.concatenate([edge_index[1], jnp.zeros((pad,), jnp.int32)]).reshape(NW, NGW, G)
    zeros2 = jnp.zeros((NPAD, 2), jnp.float32)
    zeros32 = jnp.zeros((NPAD, 32), jnp.float32)
    convs = jnp.broadcast_to(
        conv_atts.reshape(LAYER_NUM - 1, NC, 32, 1), (LAYER_NUM - 1, NC, 32, 16)
    ).astype(jnp.float32)

    b0r = b0.reshape(1, HID)
    b1r = b1.reshape(1, HID)
    ha0 = hop_att0.reshape(1, HID)
    hbs = hop_biases.reshape(LAYER_NUM, 1, HEAD)
    hatA = hop_atts[:, 0, :, :DH].reshape(LAYER_NUM - 1, 1, HID)
    hatB = hop_atts[:, 0, :, DH:].reshape(LAYER_NUM - 1, 1, HID)
    batch3 = batch.reshape(NBLK, 1, RB)

    xf, z, zs2 = _t0_call(x, W0, b0r, W1, b1r, ha0, hbs[0])
    for i in range(LAYER_NUM - 1):
        aij, adjp = _sc_pass1(srcp, dstp, zs2.reshape(2 * N, 32), convs[i], zeros2)
        adj = adjp[:, :N, :]
        xs2, isq = _t1_call(adj, xf)
        sp = _sc_pass2(srcp, dstp, xs2.reshape(2 * N, 32), aij, zeros32)
        s2 = sp[:, :N, :]
        logk_next = _LOGK[i + 1] if i + 1 < LAYER_NUM - 1 else _LOGK[-1]
        xf, z, zs2 = _t2_call(s2, isq, z, hatA[i], hatB[i], hbs[i + 1],
                              _LOGK[i], logk_next)
    return _t3_call(z, batch3, Wh, bh.reshape(1, 1))


# speedup vs baseline: 41.4007x; 1.0148x over previous
import functools

import jax
import jax.numpy as jnp
from jax import lax
from jax.experimental import pallas as pl
from jax.experimental.pallas import tpu as pltpu
from jax.experimental.pallas import tpu_sc as plsc

N = 50000
E = 800000
NUM_GRAPHS = 64
IN_CH = 128
HID = 64
HEAD = 4
DH = 16
LAYER_NUM = 10
LAMBD = 1.0

NC = 2
NS = 16
NW = NC * NS
G = 128
EPW = 25600
NGW = EPW // G
EP = NW * EPW
NPAD = 51200
RPT = NPAD // NS
RB = 1000
NBLK = N // RB

import math

_LOGK = [math.log(LAMBD / k + 1.0 + 1e-6) for k in range(1, LAYER_NUM + 1)]


def _iota16():
    return jnp.arange(16, dtype=jnp.int32)


def _elu(v):
    return jnp.where(v > 0, v, jnp.exp(v) - 1.0)


def _sc_softplus(s):
    t = jnp.exp(-jnp.abs(s))
    u = 1.0 + t
    w = t / (u + 1.0)
    w2 = w * w
    poly = 1.0 / 3.0 + w2 * (1.0 / 5.0 + w2 * (1.0 / 7.0 + w2 * (1.0 / 9.0)))
    ln_u = 2.0 * w * (1.0 + w2 * poly)
    return jnp.maximum(s, 0.0) + ln_u


IDXC = 20
NSG = NGW // IDXC


def _adjust_indices(raw, adj, cn, ngroups):
    def per_group(g, carry):
        for j in range(G // 16):
            sl = pl.ds(j * 16, 16)
            adj[g, sl] = raw[g, sl] + cn
        return carry
    lax.fori_loop(0, ngroups, per_group, 0)


def _sc1_body(src_h, dst_h, zs_h, conv_h, zeros2_h,
              aij_h, adj_h,
              idxs, idxd, idxr, rs2, rd2, convv, av2, adj_s, sem, semw):
    c = lax.axis_index("c")
    s = lax.axis_index("s")
    w = s * NC + c
    cn = c * N

    pltpu.sync_copy(zeros2_h.at[pl.ds(s * RPT, RPT)], adj_s.at[pl.ds(s * RPT, RPT)])
    pltpu.sync_copy(conv_h.at[c], convv)
    cvs = [convv[ch, :] for ch in range(32)]
    plsc.subcore_barrier()

    def supergroup(t, carry0):
        pltpu.sync_copy(src_h.at[w, pl.ds(t * IDXC, IDXC)], idxs)
        pltpu.sync_copy(dst_h.at[w, pl.ds(t * IDXC, IDXC)], idxr)
        _adjust_indices(idxs, idxs, cn, IDXC)
        _adjust_indices(idxr, idxd, cn, IDXC)
        pltpu.async_copy(zs_h.at[idxs.at[0]], rs2.at[0], sem.at[0])
        pltpu.async_copy(zs_h.at[idxd.at[0]], rd2.at[0], sem.at[0])

        def group(g, carry):
            p = lax.rem(g, 2)
            q = 1 - p

            @pl.when(g + 1 < IDXC)
            def _prefetch():
                pltpu.async_copy(zs_h.at[idxs.at[g + 1]], rs2.at[q], sem.at[q])
                pltpu.async_copy(zs_h.at[idxd.at[g + 1]], rd2.at[q], sem.at[q])

            pltpu.make_async_copy(zs_h.at[pl.ds(0, G)], rs2.at[p], sem.at[p]).wait()
            pltpu.make_async_copy(zs_h.at[pl.ds(0, G)], rd2.at[p], sem.at[p]).wait()

            @pl.when(t * IDXC + g >= 2)
            def _drainw():
                pltpu.make_async_copy(av2.at[p], aij_h.at[pl.ds(0, G)], semw.at[p]).wait()

            pv = jnp.full((16,), p, jnp.int32)
            base = w * EPW + t * IDXC * G + g * G
            for k in range(G // 16):
                rows = _iota16() + (k * 16)
                eid = _iota16() + (base + k * 16)
                mask = eid < E
                acc0 = jnp.zeros((16,), jnp.float32)
                acc1 = jnp.zeros((16,), jnp.float32)
                for ch in range(32):
                    col = jnp.full((16,), ch, jnp.int32)
                    v = (plsc.load_gather(rs2, [pv, rows, col])
                         + plsc.load_gather(rd2, [pv, rows, col]))
                    ev = jnp.where(v > 0, v, jnp.exp(v) - 1.0)
                    contrib = ev * cvs[ch]
                    if ch < 16:
                        acc0 = acc0 + contrib
                    else:
                        acc1 = acc1 + contrib
                a0 = jnp.where(mask, _sc_softplus(acc0) + 1e-6, 0.0)
                a1 = jnp.where(mask, _sc_softplus(acc1) + 1e-6, 0.0)
                plsc.store_scatter(av2, [pv, rows, jnp.zeros((16,), jnp.int32)], a0)
                plsc.store_scatter(av2, [pv, rows, jnp.ones((16,), jnp.int32)], a1)
            pltpu.async_copy(av2.at[p], aij_h.at[pl.ds(c * EP + base, G)], semw.at[p])
            pltpu.sync_copy(av2.at[p], adj_s.at[idxr.at[g]], add=True)
            return carry

        lax.fori_loop(0, IDXC, group, 0)
        return carry0

    lax.fori_loop(0, NSG, supergroup, 0)
    for p in (0, 1):
        pltpu.make_async_copy(av2.at[p], aij_h.at[pl.ds(0, G)], semw.at[p]).wait()
    plsc.subcore_barrier()
    pltpu.sync_copy(adj_s.at[pl.ds(s * RPT, RPT)], adj_h.at[c, pl.ds(s * RPT, RPT)])


_SC_PARAMS = pltpu.CompilerParams(needs_layout_passes=False, use_tc_tiling_on_sc=False)


def _sc_pass1(srcp, dstp, zs, conv, zeros2):
    mesh = plsc.VectorSubcoreMesh(core_axis_name="c", subcore_axis_name="s")
    f = pl.kernel(
        _sc1_body,
        compiler_params=_SC_PARAMS,
        out_type=(
            jax.ShapeDtypeStruct((2 * EP, 2), jnp.float32),
            jax.ShapeDtypeStruct((NC, NPAD, 2), jnp.float32),
        ),
        mesh=mesh,
        scratch_types=(
            pltpu.VMEM((IDXC, G), jnp.int32),
            pltpu.VMEM((IDXC, G), jnp.int32),
            pltpu.VMEM((IDXC, G), jnp.int32),
            pltpu.VMEM((2, G, 32), jnp.float32),
            pltpu.VMEM((2, G, 32), jnp.float32),
            pltpu.VMEM((32, 16), jnp.float32),
            pltpu.VMEM((2, G, 2), jnp.float32),
            pltpu.VMEM_SHARED((NPAD, 2), jnp.float32),
            pltpu.SemaphoreType.DMA((2,)),
            pltpu.SemaphoreType.DMA((2,)),
        ),
    )
    return f(srcp, dstp, zs, conv, zeros2)


def _sc2_body(src_h, dst_h, xs_h, aij_h, zeros32_h,
              out_h,
              idxs, idxr, rx2, ar2, msg2, out_s, sem, semw):
    c = lax.axis_index("c")
    s = lax.axis_index("s")
    w = s * NC + c
    cn = c * N

    pltpu.sync_copy(zeros32_h.at[pl.ds(s * RPT, RPT)], out_s.at[pl.ds(s * RPT, RPT)])
    plsc.subcore_barrier()

    def supergroup(t, carry0):
        pltpu.sync_copy(src_h.at[w, pl.ds(t * IDXC, IDXC)], idxs)
        pltpu.sync_copy(dst_h.at[w, pl.ds(t * IDXC, IDXC)], idxr)
        _adjust_indices(idxs, idxs, cn, IDXC)
        sbase = w * EPW + t * IDXC * G
        pltpu.async_copy(xs_h.at[idxs.at[0]], rx2.at[0], sem.at[0])
        pltpu.async_copy(aij_h.at[pl.ds(c * EP + sbase, G)], ar2.at[0], sem.at[0])

        def group(g, carry):
            p = lax.rem(g, 2)
            q = 1 - p

            @pl.when(g + 1 < IDXC)
            def _prefetch():
                pltpu.async_copy(xs_h.at[idxs.at[g + 1]], rx2.at[q], sem.at[q])
                pltpu.async_copy(aij_h.at[pl.ds(c * EP + sbase + (g + 1) * G, G)],
                                 ar2.at[q], sem.at[q])

            pltpu.make_async_copy(xs_h.at[pl.ds(0, G)], rx2.at[p], sem.at[p]).wait()
            pltpu.make_async_copy(aij_h.at[pl.ds(0, G)], ar2.at[p], sem.at[p]).wait()

            pv = jnp.full((16,), p, jnp.int32)
            for k in range(G // 16):
                rows = _iota16() + (k * 16)
                a0 = plsc.load_gather(ar2, [pv, rows, jnp.zeros((16,), jnp.int32)])
                a1 = plsc.load_gather(ar2, [pv, rows, jnp.ones((16,), jnp.int32)])
                for ch in range(32):
                    col = jnp.full((16,), ch, jnp.int32)
                    xv = plsc.load_gather(rx2, [pv, rows, col])
                    m = xv * (a0 if ch < 16 else a1)
                    plsc.store_scatter(msg2, [pv, rows, col], m)
            pltpu.sync_copy(msg2.at[p], out_s.at[idxr.at[g]], add=True)
            return carry

        lax.fori_loop(0, IDXC, group, 0)
        return carry0

    lax.fori_loop(0, NSG, supergroup, 0)
    plsc.subcore_barrier()
    pltpu.sync_copy(out_s.at[pl.ds(s * RPT, RPT)], out_h.at[c, pl.ds(s * RPT, RPT)])


def _sc_pass2(srcp, dstp, xs, aij, zeros32):
    mesh = plsc.VectorSubcoreMesh(core_axis_name="c", subcore_axis_name="s")
    f = pl.kernel(
        _sc2_body,
        compiler_params=_SC_PARAMS,
        out_type=jax.ShapeDtypeStruct((NC, NPAD, 32), jnp.float32),
        mesh=mesh,
        scratch_types=(
            pltpu.VMEM((IDXC, G), jnp.int32),
            pltpu.VMEM((IDXC, G), jnp.int32),
            pltpu.VMEM((2, G, 32), jnp.float32),
            pltpu.VMEM((2, G, 2), jnp.float32),
            pltpu.VMEM((2, G, 32), jnp.float32),
            pltpu.VMEM_SHARED((NPAD, 32), jnp.float32),
            pltpu.SemaphoreType.DMA((2,)),
            pltpu.SemaphoreType.DMA((2,)),
        ),
    )
    return f(srcp, dstp, xs, aij, zeros32)


def _head_mat():
    ci = lax.broadcasted_iota(jnp.int32, (HID, HEAD), 0) // DH
    hi = lax.broadcasted_iota(jnp.int32, (HID, HEAD), 1)
    return (ci == hi).astype(jnp.float32)


def _t0_body(x_ref, w0_ref, b0_ref, w1_ref, b1_ref, ha0_ref, hb0_ref,
             xf_ref, z_ref, zs2_ref):
    S = _head_mat()
    h = jnp.dot(x_ref[...], w0_ref[...], preferred_element_type=jnp.float32) + b0_ref[...]
    h = _elu(h)
    h = jnp.dot(h, w1_ref[...], preferred_element_type=jnp.float32) + b1_ref[...]
    xf_ref[...] = h
    att = jnp.dot(_elu(h) * ha0_ref[...], S, preferred_element_type=jnp.float32) + hb0_ref[...]
    z = h * jnp.dot(att, S.T, preferred_element_type=jnp.float32)
    z_ref[...] = z
    zs = z * _LOGK[0]
    zs2_ref[0] = zs[:, :32]
    zs2_ref[1] = zs[:, 32:]


def _t0_call(x, W0, b0, W1, b1, ha0, hb0):
    return pl.pallas_call(
        _t0_body,
        grid=(NBLK,),
        in_specs=[
            pl.BlockSpec((RB, IN_CH), lambda i: (i, 0)),
            pl.BlockSpec((IN_CH, HID), lambda i: (0, 0)),
            pl.BlockSpec((1, HID), lambda i: (0, 0)),
            pl.BlockSpec((HID, HID), lambda i: (0, 0)),
            pl.BlockSpec((1, HID), lambda i: (0, 0)),
            pl.BlockSpec((1, HID), lambda i: (0, 0)),
            pl.BlockSpec((1, HEAD), lambda i: (0, 0)),
        ],
        out_specs=[
            pl.BlockSpec((RB, HID), lambda i: (i, 0)),
            pl.BlockSpec((RB, HID), lambda i: (i, 0)),
            pl.BlockSpec((2, RB, 32), lambda i: (0, i, 0)),
        ],
        out_shape=[
            jax.ShapeDtypeStruct((N, HID), jnp.float32),
            jax.ShapeDtypeStruct((N, HID), jnp.float32),
            jax.ShapeDtypeStruct((2, N, 32), jnp.float32),
        ],
    )(x, W0, b0, W1, b1, ha0, hb0)


def _t1_body(adj_ref, xf_ref, xs2_ref, isq_ref):
    S = _head_mat()
    adj = jnp.concatenate([adj_ref[0], adj_ref[1]], axis=1)
    isq = jax.lax.rsqrt(jnp.maximum(adj, 1e-32))
    isq_ref[...] = isq
    xs = xf_ref[...] * jnp.dot(isq, S.T, preferred_element_type=jnp.float32)
    xs2_ref[0] = xs[:, :32]
    xs2_ref[1] = xs[:, 32:]


def _t1_call(adj, xf):
    return pl.pallas_call(
        _t1_body,
        grid=(NBLK,),
        in_specs=[
            pl.BlockSpec((2, RB, 2), lambda i: (0, i, 0)),
            pl.BlockSpec((RB, HID), lambda i: (i, 0)),
        ],
        out_specs=[
            pl.BlockSpec((2, RB, 32), lambda i: (0, i, 0)),
            pl.BlockSpec((RB, HEAD), lambda i: (i, 0)),
        ],
        out_shape=[
            jax.ShapeDtypeStruct((2, N, 32), jnp.float32),
            jax.ShapeDtypeStruct((N, HEAD), jnp.float32),
        ],
    )(adj, xf)


def _t2_body(s2_ref, isq_ref, z_ref, hatA_ref, hatB_ref, hb_ref,
             xf_ref, zn_ref, zs2_ref, *, logk, logk_next):
    S = _head_mat()
    sfull = jnp.concatenate([s2_ref[0], s2_ref[1]], axis=1)
    xf = sfull * jnp.dot(isq_ref[...], S.T, preferred_element_type=jnp.float32)
    xf_ref[...] = xf
    z = z_ref[...]
    zs = z * logk
    hop = (jnp.dot(_elu(xf) * hatA_ref[...], S, preferred_element_type=jnp.float32)
           + jnp.dot(_elu(zs) * hatB_ref[...], S, preferred_element_type=jnp.float32)
           + hb_ref[...])
    zn = z + xf * jnp.dot(hop, S.T, preferred_element_type=jnp.float32)
    zn_ref[...] = zn
    zsn = zn * logk_next
    zs2_ref[0] = zsn[:, :32]
    zs2_ref[1] = zsn[:, 32:]


def _t2_call(s2, isq, z, hatA, hatB, hb, logk, logk_next):
    body = functools.partial(_t2_body, logk=logk, logk_next=logk_next)
    return pl.pallas_call(
        body,
        grid=(NBLK,),
        in_specs=[
            pl.BlockSpec((2, RB, 32), lambda i: (0, i, 0)),
            pl.BlockSpec((RB, HEAD), lambda i: (i, 0)),
            pl.BlockSpec((RB, HID), lambda i: (i, 0)),
            pl.BlockSpec((1, HID), lambda i: (0, 0)),
            pl.BlockSpec((1, HID), lambda i: (0, 0)),
            pl.BlockSpec((1, HEAD), lambda i: (0, 0)),
        ],
        out_specs=[
            pl.BlockSpec((RB, HID), lambda i: (i, 0)),
            pl.BlockSpec((RB, HID), lambda i: (i, 0)),
            pl.BlockSpec((2, RB, 32), lambda i: (0, i, 0)),
        ],
        out_shape=[
            jax.ShapeDtypeStruct((N, HID), jnp.float32),
            jax.ShapeDtypeStruct((N, HID), jnp.float32),
            jax.ShapeDtypeStruct((2, N, 32), jnp.float32),
        ],
    )(s2, isq, z, hatA, hatB, hb)


def _t3_body(z_ref, batch_ref, wh_ref, bh_ref, sums_ref, cnts_ref, y_ref):
    i = pl.program_id(0)
    zf = _elu(z_ref[...])
    b = batch_ref[0]
    oh = (lax.broadcasted_iota(jnp.int32, (NUM_GRAPHS, RB), 0) == b).astype(jnp.float32)
    psum = jnp.dot(oh, zf, preferred_element_type=jnp.float32)
    pcnt = jnp.dot(oh, jnp.ones((RB, HID), jnp.float32), preferred_element_type=jnp.float32)

    @pl.when(i == 0)
    def _init():
        sums_ref[...] = psum
        cnts_ref[...] = pcnt

    @pl.when(i > 0)
    def _acc():
        sums_ref[...] += psum
        cnts_ref[...] += pcnt

    @pl.when(i == NBLK - 1)
    def _fin():
        pooled = sums_ref[...] / jnp.maximum(cnts_ref[...], 1.0)
        y_ref[...] = jnp.dot(pooled, wh_ref[...], preferred_element_type=jnp.float32) + bh_ref[...]


def _t3_call(z, batch3, Wh, bh):
    outs = pl.pallas_call(
        _t3_body,
        grid=(NBLK,),
        in_specs=[
            pl.BlockSpec((RB, HID), lambda i: (i, 0)),
            pl.BlockSpec((1, 1, RB), lambda i: (i, 0, 0)),
            pl.BlockSpec((HID, 1), lambda i: (0, 0)),
            pl.BlockSpec((1, 1), lambda i: (0, 0)),
        ],
        out_specs=[
            pl.BlockSpec((NUM_GRAPHS, HID), lambda i: (0, 0)),
            pl.BlockSpec((NUM_GRAPHS, HID), lambda i: (0, 0)),
            pl.BlockSpec((NUM_GRAPHS, 1), lambda i: (0, 0)),
        ],
        out_shape=[
            jax.ShapeDtypeStruct((NUM_GRAPHS, HID), jnp.float32),
            jax.ShapeDtypeStruct((NUM_GRAPHS, HID), jnp.float32),
            jax.ShapeDtypeStruct((NUM_GRAPHS, 1), jnp.float32),
        ],
    )(z, batch3, Wh, bh)
    return outs[2]


def kernel(x, edge_index, batch, W0, b0, W1, b1, hop_att0, hop_atts,
           hop_biases, conv_atts, Wh, bh):
    pad = EP - E
    srcp = jnp.concatenate([edge_index[0], jnp.zeros((pad,), jnp.int32)]).reshape(NW, NGW, G)
    dstp = jnp.concatenate([edge_index[1], jnp.zeros((pad,), jnp.int32)]).reshape(NW, NGW, G)
    zeros2 = jnp.zeros((NPAD, 2), jnp.float32)
    zeros32 = jnp.zeros((NPAD, 32), jnp.float32)
    convs = jnp.broadcast_to(
        conv_atts.reshape(LAYER_NUM - 1, NC, 32, 1), (LAYER_NUM - 1, NC, 32, 16)
    ).astype(jnp.float32)

    b0r = b0.reshape(1, HID)
    b1r = b1.reshape(1, HID)
    ha0 = hop_att0.reshape(1, HID)
    hbs = hop_biases.reshape(LAYER_NUM, 1, HEAD)
    hatA = hop_atts[:, 0, :, :DH].reshape(LAYER_NUM - 1, 1, HID)
    hatB = hop_atts[:, 0, :, DH:].reshape(LAYER_NUM - 1, 1, HID)
    batch3 = batch.reshape(NBLK, 1, RB)

    xf, z, zs2 = _t0_call(x, W0, b0r, W1, b1r, ha0, hbs[0])
    for i in range(LAYER_NUM - 1):
        aij, adjp = _sc_pass1(srcp, dstp, zs2.reshape(2 * N, 32), convs[i], zeros2)
        adj = adjp[:, :N, :]
        xs2, isq = _t1_call(adj, xf)
        sp = _sc_pass2(srcp, dstp, xs2.reshape(2 * N, 32), aij, zeros32)
        s2 = sp[:, :N, :]
        logk_next = _LOGK[i + 1] if i + 1 < LAYER_NUM - 1 else _LOGK[-1]
        xf, z, zs2 = _t2_call(s2, isq, z, hatA[i], hatB[i], hbs[i + 1],
                              _LOGK[i], logk_next)
    return _t3_call(z, batch3, Wh, bh.reshape(1, 1))
